# baseline (speedup 1.0000x reference)
import functools

import numpy as np
import jax
import jax.numpy as jnp
from jax import lax
from jax.experimental import pallas as pl
from jax.experimental.pallas import tpu as pltpu
from jax.experimental.pallas import tpu_sc as plsc

HIDDEN = 64

_j = np.arange(HIDDEN)
_ang = 2.0 * np.pi * np.outer(_j, _j) / HIDDEN
_WR = np.cos(_ang).astype(np.float32)
_WI = (-np.sin(_ang)).astype(np.float32)


@functools.cache
def _make_sc_gather(B: int, ENT: int, REL: int):
    info = plsc.get_sparse_core_info()
    NC, NS = info.num_cores, info.num_subcores
    NW = NC * NS
    assert B % NW == 0
    bpw = B // NW
    mesh = plsc.VectorSubcoreMesh(core_axis_name="c", subcore_axis_name="s")

    @functools.partial(
        pl.kernel,
        mesh=mesh,
        out_type=(
            jax.ShapeDtypeStruct((B, HIDDEN), jnp.float32),
            jax.ShapeDtypeStruct((B, HIDDEN), jnp.float32),
            jax.ShapeDtypeStruct((B, HIDDEN), jnp.float32),
        ),
        scratch_types=[
            pltpu.SMEM((bpw,), jnp.int32),
            pltpu.SMEM((bpw,), jnp.int32),
            pltpu.SMEM((bpw,), jnp.int32),
            pltpu.VMEM_SHARED((NS, 3 * bpw), jnp.int32),
            pltpu.SemaphoreType.DMA,
        ],
    )
    def sc_gather(h_hbm, t_hbm, r_hbm, ent_hbm, rel_hbm,
                  oh, ot, orel, hi_s, ti_s, ri_s, idx_sh, sem):
        cid = lax.axis_index("c")
        sid = lax.axis_index("s")
        wid = sid * NC + cid
        base = wid * bpw
        mine = idx_sh.at[sid]
        pltpu.sync_copy(h_hbm.at[pl.ds(base, bpw)], mine.at[pl.ds(0, bpw)])
        pltpu.sync_copy(t_hbm.at[pl.ds(base, bpw)], mine.at[pl.ds(bpw, bpw)])
        pltpu.sync_copy(r_hbm.at[pl.ds(base, bpw)], mine.at[pl.ds(2 * bpw, bpw)])
        pltpu.sync_copy(mine.at[pl.ds(0, bpw)], hi_s)
        pltpu.sync_copy(mine.at[pl.ds(bpw, bpw)], ti_s)
        pltpu.sync_copy(mine.at[pl.ds(2 * bpw, bpw)], ri_s)

        def fire(i, _):
            pltpu.async_copy(
                ent_hbm.at[pl.ds(hi_s[i], 1)], oh.at[pl.ds(base + i, 1)], sem)
            pltpu.async_copy(
                ent_hbm.at[pl.ds(ti_s[i], 1)], ot.at[pl.ds(base + i, 1)], sem)
            pltpu.async_copy(
                rel_hbm.at[pl.ds(ri_s[i], 1)], orel.at[pl.ds(base + i, 1)], sem)
            return 0

        lax.fori_loop(0, bpw, fire, 0)
        pltpu.make_async_copy(
            ent_hbm.at[pl.ds(0, bpw)], oh.at[pl.ds(base, bpw)], sem).wait()
        pltpu.make_async_copy(
            ent_hbm.at[pl.ds(0, bpw)], ot.at[pl.ds(base, bpw)], sem).wait()
        pltpu.make_async_copy(
            rel_hbm.at[pl.ds(0, bpw)], orel.at[pl.ds(base, bpw)], sem).wait()

    return sc_gather


def _tc_body(h_ref, t_ref, r_ref, wr_ref, wi_ref, out_ref):
    f32 = jnp.float32
    hp = jax.lax.Precision.HIGHEST
    h = h_ref[...]
    t = t_ref[...]
    r = r_ref[...]
    wr = wr_ref[...]
    wi = wi_ref[...]
    hr = jnp.dot(h, wr, precision=hp, preferred_element_type=f32)
    hi = jnp.dot(h, wi, precision=hp, preferred_element_type=f32)
    tr = jnp.dot(t, wr, precision=hp, preferred_element_type=f32)
    ti = jnp.dot(t, wi, precision=hp, preferred_element_type=f32)
    rr = jnp.dot(r, wr, precision=hp, preferred_element_type=f32)
    ri = jnp.dot(r, wi, precision=hp, preferred_element_type=f32)
    p = (hr * tr + hi * ti) * rr + (hr * ti - hi * tr) * ri
    s = jnp.sum(p, axis=1, keepdims=True) * (1.0 / HIDDEN)
    nrm = lax.rsqrt(jnp.maximum(jnp.sum(r * r, axis=1, keepdims=True), 1e-12))
    out_ref[...] = -jax.nn.sigmoid(s * nrm)


def _tc_score(h_e, t_e, r_e, interpret=False):
    B = h_e.shape[0]
    BLK = min(B, 2048)
    assert B % BLK == 0
    wr = jnp.asarray(_WR)
    wi = jnp.asarray(_WI)
    return pl.pallas_call(
        _tc_body,
        grid=(B // BLK,),
        in_specs=[
            pl.BlockSpec((BLK, HIDDEN), lambda i: (i, 0)),
            pl.BlockSpec((BLK, HIDDEN), lambda i: (i, 0)),
            pl.BlockSpec((BLK, HIDDEN), lambda i: (i, 0)),
            pl.BlockSpec((HIDDEN, HIDDEN), lambda i: (0, 0)),
            pl.BlockSpec((HIDDEN, HIDDEN), lambda i: (0, 0)),
        ],
        out_specs=pl.BlockSpec((BLK, 1), lambda i: (i, 0)),
        out_shape=jax.ShapeDtypeStruct((B, 1), jnp.float32),
        interpret=interpret,
    )(h_e, t_e, r_e, wr, wi)


def kernel(h, t, r, ent_embeddings, rel_embeddings):
    h = h.astype(jnp.int32)
    t = t.astype(jnp.int32)
    r = r.astype(jnp.int32)
    B = h.shape[0]
    gather = _make_sc_gather(B, ent_embeddings.shape[0], rel_embeddings.shape[0])
    h_e, t_e, r_e = gather(h, t, r, ent_embeddings, rel_embeddings)
    return _tc_score(h_e, t_e, r_e)


# speedup vs baseline: 1.7043x; 1.7043x over previous
import functools

import numpy as np
import jax
import jax.numpy as jnp
from jax import lax
from jax.experimental import pallas as pl
from jax.experimental.pallas import tpu as pltpu
from jax.experimental.pallas import tpu_sc as plsc

HIDDEN = 64
ROWW = 2 * HIDDEN

_j = np.arange(HIDDEN)
_ang = 2.0 * np.pi * np.outer(_j, _j) / HIDDEN
_WR = np.cos(_ang).astype(np.float32)
_WI = (-np.sin(_ang)).astype(np.float32)


@functools.cache
def _make_sc_gather(B: int):
    info = plsc.get_sparse_core_info()
    NC, NS = info.num_cores, info.num_subcores
    NW = NC * NS
    assert B % NW == 0
    bpw = B // NW
    CH = 128
    assert bpw % CH == 0
    nch = bpw // CH
    mesh = plsc.VectorSubcoreMesh(core_axis_name="c", subcore_axis_name="s")

    @functools.partial(
        pl.kernel,
        mesh=mesh,
        out_type=(
            jax.ShapeDtypeStruct((B, ROWW), jnp.float32),
            jax.ShapeDtypeStruct((B, ROWW), jnp.float32),
            jax.ShapeDtypeStruct((B, ROWW), jnp.float32),
        ),
        scratch_types=[
            pltpu.VMEM((bpw,), jnp.int32),
            pltpu.VMEM((bpw,), jnp.int32),
            pltpu.VMEM((bpw,), jnp.int32),
            pltpu.VMEM((2, CH, ROWW), jnp.float32),
            pltpu.VMEM((2, CH, ROWW), jnp.float32),
            pltpu.VMEM((2, CH, ROWW), jnp.float32),
            pltpu.SemaphoreType.DMA,
            pltpu.SemaphoreType.DMA,
            pltpu.SemaphoreType.DMA,
        ],
    )
    def sc_gather(h_hbm, t_hbm, r_hbm, ent2_hbm, rel2_hbm,
                  oh, ot, orel, hi_v, ti_v, ri_v, hbuf, tbuf, rbuf,
                  gsem0, gsem1, wsem):
        wid = lax.axis_index("s") * NC + lax.axis_index("c")
        base = wid * bpw
        pltpu.sync_copy(h_hbm.at[pl.ds(base, bpw)], hi_v)
        pltpu.sync_copy(t_hbm.at[pl.ds(base, bpw)], ti_v)
        pltpu.sync_copy(r_hbm.at[pl.ds(base, bpw)], ri_v)
        gsems = (gsem0, gsem1)

        def fire(c, slot):
            sl = pl.ds(c * CH, CH)
            sem = gsems[slot]
            pltpu.async_copy(ent2_hbm.at[hi_v.at[sl]], hbuf.at[slot], sem)
            pltpu.async_copy(ent2_hbm.at[ti_v.at[sl]], tbuf.at[slot], sem)
            pltpu.async_copy(rel2_hbm.at[ri_v.at[sl]], rbuf.at[slot], sem)

        def drain_gather(slot):
            sem = gsems[slot]
            pltpu.make_async_copy(
                ent2_hbm.at[pl.ds(0, CH)], hbuf.at[slot], sem).wait()
            pltpu.make_async_copy(
                ent2_hbm.at[pl.ds(0, CH)], tbuf.at[slot], sem).wait()
            pltpu.make_async_copy(
                rel2_hbm.at[pl.ds(0, CH)], rbuf.at[slot], sem).wait()

        def write(c, slot):
            osl = pl.ds(base + c * CH, CH)
            pltpu.async_copy(hbuf.at[slot], oh.at[osl], wsem)
            pltpu.async_copy(tbuf.at[slot], ot.at[osl], wsem)
            pltpu.async_copy(rbuf.at[slot], orel.at[osl], wsem)

        def drain_write(slot):
            pltpu.make_async_copy(
                ent2_hbm.at[pl.ds(0, CH)], hbuf.at[slot], wsem).wait()
            pltpu.make_async_copy(
                ent2_hbm.at[pl.ds(0, CH)], tbuf.at[slot], wsem).wait()
            pltpu.make_async_copy(
                rel2_hbm.at[pl.ds(0, CH)], rbuf.at[slot], wsem).wait()

        fire(0, 0)
        for c in range(nch):
            slot = c % 2
            nxt = 1 - slot
            if c + 1 < nch:
                if c >= 1:
                    drain_write(nxt)
                fire(c + 1, nxt)
            drain_gather(slot)
            write(c, slot)
        drain_write(0)
        drain_write(1)

    return sc_gather


def _tc_body(h_ref, t_ref, r_ref, ph_ref, pt_ref, pr_ref, wr_ref, wi_ref,
             out_ref):
    f32 = jnp.float32
    h2 = h_ref[...]
    t2 = t_ref[...]
    r2 = r_ref[...]
    ph = ph_ref[...]
    pt = pt_ref[...]
    pr = pr_ref[...]
    h = h2[:, :HIDDEN] + ph * (h2[:, HIDDEN:] - h2[:, :HIDDEN])
    t = t2[:, :HIDDEN] + pt * (t2[:, HIDDEN:] - t2[:, :HIDDEN])
    r = r2[:, :HIDDEN] + pr * (r2[:, HIDDEN:] - r2[:, :HIDDEN])
    wr = wr_ref[...]
    wi = wi_ref[...]
    hr = jnp.dot(h, wr, preferred_element_type=f32)
    hi = jnp.dot(h, wi, preferred_element_type=f32)
    tr = jnp.dot(t, wr, preferred_element_type=f32)
    ti = jnp.dot(t, wi, preferred_element_type=f32)
    rr = jnp.dot(r, wr, preferred_element_type=f32)
    ri = jnp.dot(r, wi, preferred_element_type=f32)
    p = (hr * tr + hi * ti) * rr + (hr * ti - hi * tr) * ri
    s = jnp.sum(p, axis=1, keepdims=True) * (1.0 / HIDDEN)
    nrm = lax.rsqrt(jnp.maximum(jnp.sum(r * r, axis=1, keepdims=True), 1e-12))
    out_ref[...] = -jax.nn.sigmoid(s * nrm)


def _tc_score(h2_e, t2_e, r2_e, ph, pt, pr, interpret=False):
    B = h2_e.shape[0]
    BLK = min(B, 2048)
    assert B % BLK == 0
    wr = jnp.asarray(_WR)
    wi = jnp.asarray(_WI)
    return pl.pallas_call(
        _tc_body,
        grid=(B // BLK,),
        in_specs=[
            pl.BlockSpec((BLK, ROWW), lambda i: (i, 0)),
            pl.BlockSpec((BLK, ROWW), lambda i: (i, 0)),
            pl.BlockSpec((BLK, ROWW), lambda i: (i, 0)),
            pl.BlockSpec((BLK, 1), lambda i: (i, 0)),
            pl.BlockSpec((BLK, 1), lambda i: (i, 0)),
            pl.BlockSpec((BLK, 1), lambda i: (i, 0)),
            pl.BlockSpec((HIDDEN, HIDDEN), lambda i: (0, 0)),
            pl.BlockSpec((HIDDEN, HIDDEN), lambda i: (0, 0)),
        ],
        out_specs=pl.BlockSpec((BLK, 1), lambda i: (i, 0)),
        out_shape=jax.ShapeDtypeStruct((B, 1), jnp.float32),
        interpret=interpret,
    )(h2_e, t2_e, r2_e, ph, pt, pr, wr, wi)


def kernel(h, t, r, ent_embeddings, rel_embeddings):
    h = h.astype(jnp.int32)
    t = t.astype(jnp.int32)
    r = r.astype(jnp.int32)
    B = h.shape[0]
    f32 = jnp.float32
    ent2 = ent_embeddings.reshape(ent_embeddings.shape[0] // 2, ROWW)
    rel2 = rel_embeddings.reshape(rel_embeddings.shape[0] // 2, ROWW)
    ph = (h & 1).astype(f32).reshape(B, 1)
    pt = (t & 1).astype(f32).reshape(B, 1)
    pr = (r & 1).astype(f32).reshape(B, 1)
    gather = _make_sc_gather(B)
    h2_e, t2_e, r2_e = gather(h >> 1, t >> 1, r >> 1, ent2, rel2)
    return _tc_score(h2_e, t2_e, r2_e, ph, pt, pr)


# speedup vs baseline: 3.3559x; 1.9690x over previous
import functools

import numpy as np
import jax
import jax.numpy as jnp
from jax import lax
from jax.experimental import pallas as pl
from jax.experimental.pallas import tpu as pltpu
from jax.experimental.pallas import tpu_sc as plsc

HIDDEN = 64
ROWW = 2 * HIDDEN

_j = np.arange(HIDDEN)
_ang = 2.0 * np.pi * np.outer(_j, _j) / HIDDEN
_WR = np.cos(_ang).astype(np.float32)
_WI = (-np.sin(_ang)).astype(np.float32)


@functools.cache
def _make_sc_gather(B: int):
    info = plsc.get_sparse_core_info()
    NC, NS = info.num_cores, info.num_subcores
    NW = NC * NS
    assert B % NW == 0
    bpw = B // NW
    CH = 128
    assert bpw % CH == 0
    nch = bpw // CH
    mesh = plsc.VectorSubcoreMesh(core_axis_name="c", subcore_axis_name="s")

    @functools.partial(
        pl.kernel,
        mesh=mesh,
        out_type=(
            jax.ShapeDtypeStruct((B, ROWW), jnp.float32),
            jax.ShapeDtypeStruct((B, ROWW), jnp.float32),
            jax.ShapeDtypeStruct((B, ROWW), jnp.float32),
        ),
        scratch_types=[
            pltpu.VMEM((bpw,), jnp.int32),
            pltpu.VMEM((bpw,), jnp.int32),
            pltpu.VMEM((bpw,), jnp.int32),
            pltpu.VMEM((2, CH, ROWW), jnp.float32),
            pltpu.VMEM((2, CH, ROWW), jnp.float32),
            pltpu.VMEM((2, CH, ROWW), jnp.float32),
            pltpu.SemaphoreType.DMA,
            pltpu.SemaphoreType.DMA,
            pltpu.SemaphoreType.DMA,
        ],
    )
    def sc_gather(h_hbm, t_hbm, r_hbm, ent2_hbm, rel2_hbm,
                  oh, ot, orel, hi_v, ti_v, ri_v, hbuf, tbuf, rbuf,
                  gsem0, gsem1, wsem):
        wid = lax.axis_index("s") * NC + lax.axis_index("c")
        base = wid * bpw
        pltpu.sync_copy(h_hbm.at[pl.ds(base, bpw)], hi_v)
        pltpu.sync_copy(t_hbm.at[pl.ds(base, bpw)], ti_v)
        pltpu.sync_copy(r_hbm.at[pl.ds(base, bpw)], ri_v)
        gsems = (gsem0, gsem1)

        def fire(c, slot):
            sl = pl.ds(c * CH, CH)
            sem = gsems[slot]
            pltpu.async_copy(ent2_hbm.at[hi_v.at[sl]], hbuf.at[slot], sem)
            pltpu.async_copy(ent2_hbm.at[ti_v.at[sl]], tbuf.at[slot], sem)
            pltpu.async_copy(rel2_hbm.at[ri_v.at[sl]], rbuf.at[slot], sem)

        def drain_gather(slot):
            sem = gsems[slot]
            pltpu.make_async_copy(
                ent2_hbm.at[pl.ds(0, CH)], hbuf.at[slot], sem).wait()
            pltpu.make_async_copy(
                ent2_hbm.at[pl.ds(0, CH)], tbuf.at[slot], sem).wait()
            pltpu.make_async_copy(
                rel2_hbm.at[pl.ds(0, CH)], rbuf.at[slot], sem).wait()

        def write(c, slot):
            osl = pl.ds(base + c * CH, CH)
            pltpu.async_copy(hbuf.at[slot], oh.at[osl], wsem)
            pltpu.async_copy(tbuf.at[slot], ot.at[osl], wsem)
            pltpu.async_copy(rbuf.at[slot], orel.at[osl], wsem)

        def drain_write(slot):
            pltpu.make_async_copy(
                ent2_hbm.at[pl.ds(0, CH)], hbuf.at[slot], wsem).wait()
            pltpu.make_async_copy(
                ent2_hbm.at[pl.ds(0, CH)], tbuf.at[slot], wsem).wait()
            pltpu.make_async_copy(
                rel2_hbm.at[pl.ds(0, CH)], rbuf.at[slot], wsem).wait()

        fire(0, 0)
        for c in range(nch):
            slot = c % 2
            nxt = 1 - slot
            if c + 1 < nch:
                if c >= 1:
                    drain_write(nxt)
                fire(c + 1, nxt)
            drain_gather(slot)
            write(c, slot)
        drain_write(0)
        drain_write(1)

    return sc_gather


def _repack_body(x1_ref, x2_ref, out_ref):
    x1 = x1_ref[...]
    x2 = x2_ref[...]
    out_ref[...] = jnp.concatenate([x1.T, x2.T], axis=1)


def _repack(tableT, split, nblk, blk):
    sb = split // blk
    return pl.pallas_call(
        _repack_body,
        grid=(nblk,),
        in_specs=[
            pl.BlockSpec((HIDDEN, blk), lambda i: (0, i)),
            pl.BlockSpec((HIDDEN, blk), lambda i: (0, sb + i)),
        ],
        out_specs=pl.BlockSpec((blk, ROWW), lambda i: (i, 0)),
        out_shape=jax.ShapeDtypeStruct((nblk * blk, ROWW), jnp.float32),
    )(tableT, tableT)


def _tc_body(h_ref, t_ref, r_ref, ph_ref, pt_ref, pr_ref, wr_ref, wi_ref,
             out_ref):
    f32 = jnp.float32
    h2 = h_ref[...]
    t2 = t_ref[...]
    r2 = r_ref[...]
    ph = ph_ref[...]
    pt = pt_ref[...]
    pr = pr_ref[...]
    h = h2[:, :HIDDEN] + ph * (h2[:, HIDDEN:] - h2[:, :HIDDEN])
    t = t2[:, :HIDDEN] + pt * (t2[:, HIDDEN:] - t2[:, :HIDDEN])
    r = r2[:, :HIDDEN] + pr * (r2[:, HIDDEN:] - r2[:, :HIDDEN])
    wr = wr_ref[...]
    wi = wi_ref[...]
    hr = jnp.dot(h, wr, preferred_element_type=f32)
    hi = jnp.dot(h, wi, preferred_element_type=f32)
    tr = jnp.dot(t, wr, preferred_element_type=f32)
    ti = jnp.dot(t, wi, preferred_element_type=f32)
    rr = jnp.dot(r, wr, preferred_element_type=f32)
    ri = jnp.dot(r, wi, preferred_element_type=f32)
    p = (hr * tr + hi * ti) * rr + (hr * ti - hi * tr) * ri
    s = jnp.sum(p, axis=1, keepdims=True) * (1.0 / HIDDEN)
    nrm = lax.rsqrt(jnp.maximum(jnp.sum(r * r, axis=1, keepdims=True), 1e-12))
    out_ref[...] = -jax.nn.sigmoid(s * nrm)


def _tc_score(h2_e, t2_e, r2_e, ph, pt, pr, interpret=False):
    B = h2_e.shape[0]
    BLK = min(B, 2048)
    assert B % BLK == 0
    wr = jnp.asarray(_WR)
    wi = jnp.asarray(_WI)
    return pl.pallas_call(
        _tc_body,
        grid=(B // BLK,),
        in_specs=[
            pl.BlockSpec((BLK, ROWW), lambda i: (i, 0)),
            pl.BlockSpec((BLK, ROWW), lambda i: (i, 0)),
            pl.BlockSpec((BLK, ROWW), lambda i: (i, 0)),
            pl.BlockSpec((BLK, 1), lambda i: (i, 0)),
            pl.BlockSpec((BLK, 1), lambda i: (i, 0)),
            pl.BlockSpec((BLK, 1), lambda i: (i, 0)),
            pl.BlockSpec((HIDDEN, HIDDEN), lambda i: (0, 0)),
            pl.BlockSpec((HIDDEN, HIDDEN), lambda i: (0, 0)),
        ],
        out_specs=pl.BlockSpec((BLK, 1), lambda i: (i, 0)),
        out_shape=jax.ShapeDtypeStruct((B, 1), jnp.float32),
        interpret=interpret,
    )(h2_e, t2_e, r2_e, ph, pt, pr, wr, wi)


def kernel(h, t, r, ent_embeddings, rel_embeddings):
    h = h.astype(jnp.int32)
    t = t.astype(jnp.int32)
    r = r.astype(jnp.int32)
    B = h.shape[0]
    f32 = jnp.float32
    BLK_E, SPLIT_E = 4096, 4096 * 122
    BLK_R, SPLIT_R = 512, 512
    nblk_e = 123
    assert SPLIT_E + nblk_e * BLK_E >= ent_embeddings.shape[0]
    ent2 = _repack(ent_embeddings.T, SPLIT_E, nblk_e, BLK_E)
    rel2 = _repack(rel_embeddings.T, SPLIT_R, 1, BLK_R)
    ph = (h >= SPLIT_E).astype(f32).reshape(B, 1)
    pt = (t >= SPLIT_E).astype(f32).reshape(B, 1)
    pr = (r >= SPLIT_R).astype(f32).reshape(B, 1)
    hrow = jnp.where(h < SPLIT_E, h, h - SPLIT_E)
    trow = jnp.where(t < SPLIT_E, t, t - SPLIT_E)
    rrow = jnp.where(r < SPLIT_R, r, r - SPLIT_R)
    gather = _make_sc_gather(B)
    h2_e, t2_e, r2_e = gather(hrow, trow, rrow, ent2, rel2)
    return _tc_score(h2_e, t2_e, r2_e, ph, pt, pr)
